# baseline (device time: 6417 ns/iter reference)
import jax
import jax.numpy as jnp
from jax import lax
from jax.experimental import pallas as pl
from jax.experimental.pallas import tpu as pltpu

Y_SIZE = 2


def kernel(x, gamma):
    m, n = x.shape
    n_global = n * Y_SIZE
    eps = 1e-5

    def body(x_hbm, g_hbm, out_hbm, xv, gv, ov, psum_ref, recv_ref,
             send_sem, recv_sem, cpx_sem, cpg_sem, cpo_sem):
        my_x = lax.axis_index("x")
        my_y = lax.axis_index("y")
        nbr = (my_x, 1 - my_y)

        barrier_sem = pltpu.get_barrier_semaphore()
        pl.semaphore_signal(
            barrier_sem, inc=1, device_id=nbr,
            device_id_type=pl.DeviceIdType.MESH,
        )

        cx = pltpu.make_async_copy(x_hbm, xv, cpx_sem)
        cg = pltpu.make_async_copy(g_hbm, gv, cpg_sem)
        cx.start()
        cg.start()
        cx.wait()

        xr = xv[...].reshape(m // 128, 128, n)
        psum_ref[...] = jnp.sum(xr * xr, axis=2)

        pl.semaphore_wait(barrier_sem, 1)

        rdma = pltpu.make_async_remote_copy(
            src_ref=psum_ref,
            dst_ref=recv_ref,
            send_sem=send_sem,
            recv_sem=recv_sem,
            device_id=nbr,
            device_id_type=pl.DeviceIdType.MESH,
        )
        rdma.start()

        cg.wait()
        xg = xr.astype(jnp.bfloat16) * gv[...].reshape(1, 1, n).astype(jnp.bfloat16)

        rdma.wait_recv()

        total = psum_ref[...] + recv_ref[...]
        inv = lax.rsqrt(total * (1.0 / n_global) + eps)
        inv16 = inv.astype(jnp.bfloat16)
        ov[...] = (xg * inv16[:, :, None]).reshape(m, n)

        co = pltpu.make_async_copy(ov, out_hbm, cpo_sem)
        co.start()

        rdma.wait_send()
        co.wait()

    return pl.pallas_call(
        body,
        out_shape=jax.ShapeDtypeStruct((m, n), jnp.bfloat16),
        in_specs=[
            pl.BlockSpec(memory_space=pl.ANY),
            pl.BlockSpec(memory_space=pl.ANY),
        ],
        out_specs=pl.BlockSpec(memory_space=pl.ANY),
        scratch_shapes=[
            pltpu.VMEM((m, n), jnp.float32),
            pltpu.VMEM((1, n), jnp.float32),
            pltpu.VMEM((m, n), jnp.bfloat16),
            pltpu.VMEM((m // 128, 128), jnp.float32),
            pltpu.VMEM((m // 128, 128), jnp.float32),
            pltpu.SemaphoreType.DMA,
            pltpu.SemaphoreType.DMA,
            pltpu.SemaphoreType.DMA,
            pltpu.SemaphoreType.DMA,
            pltpu.SemaphoreType.DMA,
        ],
        compiler_params=pltpu.CompilerParams(collective_id=0),
    )(x, gamma.reshape(1, n))


# device time: 6357 ns/iter; 1.0094x vs baseline; 1.0094x over previous
import jax
import jax.numpy as jnp
from jax import lax
from jax.experimental import pallas as pl
from jax.experimental.pallas import tpu as pltpu

Y_SIZE = 2


def kernel(x, gamma):
    m, n = x.shape
    n_global = n * Y_SIZE
    eps = 1e-5

    def body(x_ref, g_ref, out_ref, psum_ref, recv_ref, send_sem, recv_sem):
        my_x = lax.axis_index("x")
        my_y = lax.axis_index("y")
        nbr = (my_x, 1 - my_y)

        barrier_sem = pltpu.get_barrier_semaphore()
        pl.semaphore_signal(
            barrier_sem, inc=1, device_id=nbr,
            device_id_type=pl.DeviceIdType.MESH,
        )

        xr = x_ref[...].reshape(m // 128, 128, n)
        psum_ref[...] = jnp.sum(xr * xr, axis=2)

        pl.semaphore_wait(barrier_sem, 1)

        rdma = pltpu.make_async_remote_copy(
            src_ref=psum_ref,
            dst_ref=recv_ref,
            send_sem=send_sem,
            recv_sem=recv_sem,
            device_id=nbr,
            device_id_type=pl.DeviceIdType.MESH,
        )
        rdma.start()

        xg = xr.astype(jnp.bfloat16) * g_ref[...].reshape(1, 1, n).astype(jnp.bfloat16)

        rdma.wait_recv()

        total = psum_ref[...] + recv_ref[...]
        inv = lax.rsqrt(total * (1.0 / n_global) + eps)
        inv16 = inv.astype(jnp.bfloat16)
        out_ref[...] = (xg * inv16[:, :, None]).reshape(m, n)

        rdma.wait_send()

    return pl.pallas_call(
        body,
        out_shape=jax.ShapeDtypeStruct((m, n), jnp.bfloat16),
        in_specs=[
            pl.BlockSpec(memory_space=pltpu.VMEM),
            pl.BlockSpec(memory_space=pltpu.VMEM),
        ],
        out_specs=pl.BlockSpec(memory_space=pltpu.VMEM),
        scratch_shapes=[
            pltpu.VMEM((m // 128, 128), jnp.float32),
            pltpu.VMEM((m // 128, 128), jnp.float32),
            pltpu.SemaphoreType.DMA,
            pltpu.SemaphoreType.DMA,
        ],
        compiler_params=pltpu.CompilerParams(collective_id=0),
    )(x, gamma.reshape(1, n))
